# Initial kernel scaffold; baseline (speedup 1.0000x reference)
import jax
import jax.numpy as jnp
from jax.experimental import pallas as pl


def kernel(x, W_e, b_e, W_d, b_d):
    raise NotImplementedError("write your pallas kernel here")


# speedup vs baseline: 10.8886x; 10.8886x over previous
import functools

import jax
import jax.numpy as jnp
from jax.experimental import pallas as pl
from jax.experimental.pallas import tpu as pltpu

_K = 32
_BR = 256
_BH = 1024


def _f32_to_key(u):
    neg = jax.lax.shift_right_arithmetic(u, 31)
    return u ^ (neg & jnp.int32(0x7FFFFFFF))


def _ksae_kernel(nbj, x_ref, we_ref, be_ref, wd_ref, bd_ref,
                 recon_ref, hs_ref, h_ref, keys_scr, thr_scr):
    j = pl.program_id(1)
    br = x_ref.shape[0]

    @pl.when(j < nbj)
    def _encoder():
        h_tile = jax.lax.dot_general(
            x_ref[...], we_ref[...], (((1,), (1,)), ((), ())),
            preferred_element_type=jnp.float32) + be_ref[...]
        h_ref[...] = h_tile
        bits = jax.lax.bitcast_convert_type(h_tile, jnp.int32)
        keys_scr[j] = _f32_to_key(bits)

    @pl.when(j == nbj - 1)
    def _threshold():
        def count_ge(cand):
            def body(t, acc):
                chunk = keys_scr[t]
                return acc + jnp.sum((chunk >= cand).astype(jnp.int32),
                                     axis=1, keepdims=True)
            return jax.lax.fori_loop(0, nbj, body,
                                     jnp.zeros((br, 1), jnp.int32))

        cnt0 = count_ge(jnp.zeros((br, 1), jnp.int32))
        t = jnp.where(cnt0 >= _K, jnp.int32(0), jnp.int32(-2147483648))

        def bit_body(i, t):
            bit = jax.lax.shift_left(jnp.int32(1), 30 - i)
            cand = t | bit
            cnt = count_ge(cand)
            return jnp.where(cnt >= _K, cand, t)

        t = jax.lax.fori_loop(0, 31, bit_body, t)
        thr_scr[...] = t

    @pl.when(j >= nbj)
    def _decoder():
        d = j - nbj
        keys = keys_scr[d]
        thr = thr_scr[...]
        mask = keys >= thr
        hvals = jax.lax.bitcast_convert_type(_f32_to_key(keys), jnp.float32)
        hs = jnp.where(mask, hvals, jnp.float32(0.0))
        hs_ref[...] = hs
        part = jax.lax.dot_general(
            hs, wd_ref[...], (((1,), (1,)), ((), ())),
            preferred_element_type=jnp.float32)

        @pl.when(d == 0)
        def _():
            recon_ref[...] = part + bd_ref[...]

        @pl.when(d > 0)
        def _():
            recon_ref[...] += part


def kernel(x, W_e, b_e, W_d, b_d):
    B, D = x.shape
    H = W_e.shape[0]
    br = _BR if B % _BR == 0 else B
    bh = _BH if H % _BH == 0 else H
    nbi = B // br
    nbj = H // bh

    be2 = b_e.reshape(1, H)
    bd2 = b_d.reshape(1, D)

    grid = (nbi, 2 * nbj)
    last = nbj - 1

    recon, hs, h = pl.pallas_call(
        functools.partial(_ksae_kernel, nbj),
        grid=grid,
        in_specs=[
            pl.BlockSpec((br, D), lambda i, j: (i, 0)),
            pl.BlockSpec((bh, D), lambda i, j: (jnp.minimum(j, last), 0)),
            pl.BlockSpec((1, bh), lambda i, j: (0, jnp.minimum(j, last))),
            pl.BlockSpec((D, bh),
                         lambda i, j: (0, jnp.clip(j - nbj, 0, last))),
            pl.BlockSpec((1, D), lambda i, j: (0, 0)),
        ],
        out_specs=[
            pl.BlockSpec((br, D), lambda i, j: (i, 0)),
            pl.BlockSpec((br, bh),
                         lambda i, j: (i, jnp.clip(j - nbj, 0, last))),
            pl.BlockSpec((br, bh), lambda i, j: (i, jnp.minimum(j, last))),
        ],
        out_shape=[
            jax.ShapeDtypeStruct((B, D), jnp.float32),
            jax.ShapeDtypeStruct((B, H), jnp.float32),
            jax.ShapeDtypeStruct((B, H), jnp.float32),
        ],
        scratch_shapes=[
            pltpu.VMEM((nbj, br, bh), jnp.int32),
            pltpu.VMEM((br, 1), jnp.int32),
        ],
    )(x, W_e, be2, W_d, bd2)
    return (recon, hs, h)
